# baseline (speedup 1.0000x reference)
import jax
import jax.numpy as jnp
from jax.experimental import pallas as pl
from jax.experimental.pallas import tpu as pltpu
from jax.experimental.pallas import tpu_sc as plsc

BATCH = 16384
NUM_FIELDS = 26
LATENT_DIM = 32
FEATURES = 1000000
ROW_BYTES = LATENT_DIM * 4
N = BATCH * NUM_FIELDS
WINDOW = 128
GRID = N // WINDOW

_mesh = plsc.VectorSubcoreMesh(core_axis_name="c", subcore_axis_name="s")


def _gather_rows(emb_u8, x128):
    @pl.kernel(
        out_type=jax.ShapeDtypeStruct((N, ROW_BYTES), jnp.uint8),
        mesh=_mesh,
        compiler_params=pltpu.CompilerParams(use_tc_tiling_on_sc=False),
    )
    def k(emb_hbm, i_hbm, o_hbm):
        def body(i_vmem, o_vmem):
            pltpu.sync_copy(emb_hbm.at[i_vmem.at[0]], o_vmem)

        pltpu.emit_pipeline(
            body,
            grid=(GRID,),
            in_specs=[pl.BlockSpec((1, WINDOW), lambda i: (i, 0))],
            out_specs=[pl.BlockSpec((WINDOW, ROW_BYTES), lambda i: (i, 0))],
            core_axis_name=("c", "s"),
            dimension_semantics=(pltpu.PARALLEL,),
        )(i_hbm, o_hbm)

    return k(emb_u8, x128)


def kernel(x, embedding):
    x128 = x.reshape(GRID, WINDOW).astype(jnp.int32)
    emb_u8 = jax.lax.bitcast_convert_type(embedding, jnp.uint8).reshape(
        FEATURES, ROW_BYTES
    )
    out_u8 = _gather_rows(emb_u8, x128)
    out = jax.lax.bitcast_convert_type(
        out_u8.reshape(N, LATENT_DIM, 4), jnp.float32
    )
    return out.reshape(BATCH, NUM_FIELDS, LATENT_DIM)


# speedup vs baseline: 3.6976x; 3.6976x over previous
import jax
import jax.numpy as jnp
from jax.experimental import pallas as pl
from jax.experimental.pallas import tpu as pltpu
from jax.experimental.pallas import tpu_sc as plsc

BATCH = 16384
NUM_FIELDS = 26
LATENT_DIM = 32
BS = 8
GRID = BATCH // BS

_mesh = plsc.VectorSubcoreMesh(core_axis_name="c", subcore_axis_name="s")


def kernel(x, embedding):
    @pl.kernel(
        out_type=jax.ShapeDtypeStruct(
            (BATCH, NUM_FIELDS, LATENT_DIM), jnp.float32
        ),
        mesh=_mesh,
        compiler_params=pltpu.CompilerParams(use_tc_tiling_on_sc=False),
    )
    def k(i_hbm, emb_hbm, o_hbm):
        def body(i_vmem, o_vmem):
            for j in range(BS):
                pltpu.sync_copy(emb_hbm.at[i_vmem.at[j]], o_vmem.at[j])

        pltpu.emit_pipeline(
            body,
            grid=(GRID,),
            in_specs=[pl.BlockSpec((BS, NUM_FIELDS), lambda i: (i, 0))],
            out_specs=[
                pl.BlockSpec((BS, NUM_FIELDS, LATENT_DIM), lambda i: (i, 0, 0))
            ],
            core_axis_name=("c", "s"),
            dimension_semantics=(pltpu.PARALLEL,),
        )(i_hbm, o_hbm)

    return k(x, embedding)


# speedup vs baseline: 4.6178x; 1.2489x over previous
import jax
import jax.numpy as jnp
from jax.experimental import pallas as pl
from jax.experimental.pallas import tpu as pltpu
from jax.experimental.pallas import tpu_sc as plsc

BATCH = 16384
NUM_FIELDS = 26
LATENT_DIM = 32
FEATURES = 1000000
N = BATCH * NUM_FIELDS
CHUNK = 128
SUB = 8
WINDOW = CHUNK * SUB
GRID = N // WINDOW

_mesh = plsc.VectorSubcoreMesh(core_axis_name="c", subcore_axis_name="s")


def _gather_rows(emb, x128):
    @pl.kernel(
        out_type=jax.ShapeDtypeStruct((N, LATENT_DIM), jnp.float32),
        mesh=_mesh,
        compiler_params=pltpu.CompilerParams(use_tc_tiling_on_sc=False),
    )
    def k(emb_hbm, i_hbm, o_hbm):
        def body(i_vmem, o_vmem):
            for j in range(SUB):
                pltpu.sync_copy(
                    emb_hbm.at[i_vmem.at[j]],
                    o_vmem.at[pl.ds(j * CHUNK, CHUNK)],
                )

        pltpu.emit_pipeline(
            body,
            grid=(GRID,),
            in_specs=[pl.BlockSpec((SUB, CHUNK), lambda i: (i, 0))],
            out_specs=[pl.BlockSpec((WINDOW, LATENT_DIM), lambda i: (i, 0))],
            core_axis_name=("c", "s"),
            dimension_semantics=(pltpu.PARALLEL,),
        )(i_hbm, o_hbm)

    return k(emb, x128)


def kernel(x, embedding):
    x128 = x.reshape(N // CHUNK, CHUNK).astype(jnp.int32)
    out = _gather_rows(embedding, x128)
    return out.reshape(BATCH, NUM_FIELDS, LATENT_DIM)


# speedup vs baseline: 4.9196x; 1.0654x over previous
import functools

import jax
import jax.numpy as jnp
from jax import lax
from jax.experimental import pallas as pl
from jax.experimental.pallas import tpu as pltpu
from jax.experimental.pallas import tpu_sc as plsc

BATCH = 16384
NUM_FIELDS = 26
LATENT_DIM = 32
FEATURES = 1000000
N = BATCH * NUM_FIELDS
CHUNK = 128
ROWS_PER_STEP = 13
STEP = ROWS_PER_STEP * CHUNK
NWORKERS = 32
XROWS = N // CHUNK
STEPS = XROWS // (ROWS_PER_STEP * NWORKERS)

_mesh = plsc.VectorSubcoreMesh(core_axis_name="c", subcore_axis_name="s")


def _gather_rows(emb, x128):
    @functools.partial(
        pl.kernel,
        out_type=jax.ShapeDtypeStruct((N, LATENT_DIM), jnp.float32),
        mesh=_mesh,
        scratch_types=[
            pltpu.VMEM((2, ROWS_PER_STEP, CHUNK), jnp.int32),
            pltpu.VMEM((2, STEP, LATENT_DIM), jnp.float32),
            pltpu.SemaphoreType.DMA,
            pltpu.SemaphoreType.DMA,
            pltpu.SemaphoreType.DMA,
        ],
        compiler_params=pltpu.CompilerParams(use_tc_tiling_on_sc=False),
    )
    def k(emb_hbm, i_hbm, o_hbm, idx_v, rows_v, isem, gsem, osem):
        wid = lax.axis_index("s") * 2 + lax.axis_index("c")
        base = wid * (ROWS_PER_STEP * STEPS)

        def fetch(step, slot):
            pltpu.async_copy(
                i_hbm.at[pl.ds(base + step * ROWS_PER_STEP, ROWS_PER_STEP)],
                idx_v.at[slot],
                isem,
            )

        fetch(0, 0)

        @pl.loop(0, STEPS)
        def _(step):
            slot = lax.rem(step, 2)
            pltpu.make_async_copy(
                i_hbm.at[pl.ds(0, ROWS_PER_STEP)], idx_v.at[0], isem
            ).wait()

            @pl.when(step + 1 < STEPS)
            def _():
                fetch(step + 1, 1 - slot)

            @pl.when(step >= 2)
            def _():
                pltpu.make_async_copy(
                    rows_v.at[0], o_hbm.at[pl.ds(0, STEP)], osem
                ).wait()

            @pl.loop(0, ROWS_PER_STEP)
            def _(j):
                pltpu.async_copy(
                    emb_hbm.at[idx_v.at[slot, j]],
                    rows_v.at[slot, pl.ds(j * CHUNK, CHUNK)],
                    gsem,
                )

            @pl.loop(0, ROWS_PER_STEP)
            def _(j):
                pltpu.make_async_copy(
                    emb_hbm.at[idx_v.at[slot, 0]],
                    rows_v.at[slot, pl.ds(0, CHUNK)],
                    gsem,
                ).wait()

            pltpu.async_copy(
                rows_v.at[slot],
                o_hbm.at[pl.ds((base + step * ROWS_PER_STEP) * CHUNK, STEP)],
                osem,
            )

        @pl.loop(0, 2)
        def _(step):
            pltpu.make_async_copy(
                rows_v.at[0], o_hbm.at[pl.ds(0, STEP)], osem
            ).wait()

    return k(emb, x128)


def kernel(x, embedding):
    x128 = x.reshape(XROWS, CHUNK).astype(jnp.int32)
    out = _gather_rows(embedding, x128)
    return out.reshape(BATCH, NUM_FIELDS, LATENT_DIM)
